# Initial kernel scaffold; baseline (speedup 1.0000x reference)
import jax
import jax.numpy as jnp
from jax.experimental import pallas as pl


def kernel(x, layer1, layer2, layer3, layer4):
    raise NotImplementedError("write your pallas kernel here")


# speedup vs baseline: 56.7987x; 56.7987x over previous
import functools

import jax
import jax.numpy as jnp
from jax import lax
from jax.experimental import pallas as pl
from jax.experimental.pallas import tpu as pltpu
from jax.experimental.pallas import tpu_sc as plsc

_W = 4096
_H = 4096
_N_PTS = 1048576
_LVL_S = (4096, 2048, 1024, 512)
_NW = 32
_P_TILE = _N_PTS // _NW
_C = 2048
_G = _C // 16
_N_CHUNK = _P_TILE // _C
_IDX_N = 16 * _C
_ROWS_PER_LVL = 4 * _C // 128


def _tex_body(x0_hbm, x1_hbm, t0, t1, t2, t3, out_hbm,
              x0_v, x1_v, idx_v, val_v, fx_v, fxe_v, fy_v, fye_v, acc_v, sem):
    cid = lax.axis_index("c")
    sid = lax.axis_index("s")
    wid = sid * 2 + cid
    texs = (t0, t1, t2, t3)
    base = wid * _P_TILE

    def chunk_body(ci, carry):
        off = base + ci * _C
        pltpu.sync_copy(x0_hbm.at[pl.ds(off, _C)], x0_v)
        pltpu.sync_copy(x1_hbm.at[pl.ds(off, _C)], x1_v)

        def grp_body(g, c2):
            p = g * 16
            xv = x0_v[pl.ds(p, 16)]
            yv = x1_v[pl.ds(p, 16)]
            xsx = xv * 0.5 + 0.5
            xsy = yv * 0.5 + 0.5
            for l, s in enumerate(_LVL_S):
                sf = float(s)
                ix = ((xsx + 1.0) * sf - 1.0) * 0.5
                iy = ((xsy + 1.0) * sf - 1.0) * 0.5
                ix0 = ix.astype(jnp.int32)
                iy0 = iy.astype(jnp.int32)
                fx = ix - ix0.astype(jnp.float32)
                fy = iy - iy0.astype(jnp.float32)
                okx = ix0 < (s - 1)
                oky = iy0 < (s - 1)
                fxe = jnp.where(okx, fx, 0.0)
                fye = jnp.where(oky, fy, 0.0)
                ix1 = jnp.minimum(ix0 + 1, s - 1)
                iy1 = jnp.minimum(iy0 + 1, s - 1)
                r0 = iy0 * s
                r1 = iy1 * s
                lb = l * 4 * _C
                idx_v[pl.ds(lb + p, 16)] = r0 + ix0
                idx_v[pl.ds(lb + _C + p, 16)] = r0 + ix1
                idx_v[pl.ds(lb + 2 * _C + p, 16)] = r1 + ix0
                idx_v[pl.ds(lb + 3 * _C + p, 16)] = r1 + ix1
                fx_v[pl.ds(l * _C + p, 16)] = fx
                fxe_v[pl.ds(l * _C + p, 16)] = fxe
                fy_v[pl.ds(l * _C + p, 16)] = fy
                fye_v[pl.ds(l * _C + p, 16)] = fye
            return c2

        lax.fori_loop(0, _G, grp_body, 0)

        for l in range(4):
            texl = texs[l]
            lb = l * 4 * _C

            def fire_body(j, c2, texl=texl, lb=lb):
                o = lb + j * 128
                pltpu.async_copy(texl.at[idx_v.at[pl.ds(o, 128)]],
                                 val_v.at[pl.ds(o, 128)], sem)
                return c2

            lax.fori_loop(0, _ROWS_PER_LVL, fire_body, 0)

        def drain_body(j, c2):
            pltpu.make_async_copy(t0.at[idx_v.at[pl.ds(0, 128)]],
                                  val_v.at[pl.ds(0, 128)], sem).wait()
            return c2

        lax.fori_loop(0, 4 * _ROWS_PER_LVL, drain_body, 0)

        def acc_body(g, c2):
            p = g * 16
            total = jnp.zeros((16,), jnp.float32)
            for l in range(4):
                lb = l * 4 * _C
                fx = fx_v[pl.ds(l * _C + p, 16)]
                fxe = fxe_v[pl.ds(l * _C + p, 16)]
                fy = fy_v[pl.ds(l * _C + p, 16)]
                fye = fye_v[pl.ds(l * _C + p, 16)]
                v00 = val_v[pl.ds(lb + p, 16)]
                v01 = val_v[pl.ds(lb + _C + p, 16)]
                v10 = val_v[pl.ds(lb + 2 * _C + p, 16)]
                v11 = val_v[pl.ds(lb + 3 * _C + p, 16)]
                wx0 = 1.0 - fx
                wy0 = 1.0 - fy
                top = v00 * wx0 + v01 * fxe
                bot = v10 * wx0 + v11 * fxe
                total = total + (top * wy0 + bot * fye)
            acc_v[pl.ds(p, 16)] = total
            return c2

        lax.fori_loop(0, _G, acc_body, 0)
        pltpu.sync_copy(acc_v, out_hbm.at[pl.ds(off, _C)])
        return carry

    lax.fori_loop(0, _N_CHUNK, chunk_body, 0)


_mesh = plsc.VectorSubcoreMesh(core_axis_name="c", subcore_axis_name="s")

_tex_call = functools.partial(
    pl.kernel,
    mesh=_mesh,
    out_type=jax.ShapeDtypeStruct((_N_PTS,), jnp.float32),
    scratch_types=[
        pltpu.VMEM((_C,), jnp.float32),
        pltpu.VMEM((_C,), jnp.float32),
        pltpu.VMEM((_IDX_N,), jnp.int32),
        pltpu.VMEM((_IDX_N,), jnp.float32),
        pltpu.VMEM((4 * _C,), jnp.float32),
        pltpu.VMEM((4 * _C,), jnp.float32),
        pltpu.VMEM((4 * _C,), jnp.float32),
        pltpu.VMEM((4 * _C,), jnp.float32),
        pltpu.VMEM((_C,), jnp.float32),
        pltpu.SemaphoreType.DMA,
    ],
)(_tex_body)


def kernel(x, layer1, layer2, layer3, layer4):
    x0 = x[:, 0] + 0.0
    x1 = x[:, 1] + 0.0
    return _tex_call(x0, x1,
                     layer1.reshape(-1), layer2.reshape(-1),
                     layer3.reshape(-1), layer4.reshape(-1))
